# baseline (speedup 1.0000x reference)
import functools

import jax
import jax.numpy as jnp
from jax import lax
from jax.experimental import pallas as pl
from jax.experimental.pallas import tpu as pltpu
from jax.experimental.pallas import tpu_sc as plsc

B = 16384
D = 64
DP = 128
NROWS = 1000000
NPAIR = NROWS // 2
N_ACT = 300
N_PAD = 384
O_PAD = 128
Y_LO, Y_HI = 0.0, 5.5

_info = plsc.get_sparse_core_info()
NC, NS = _info.num_cores, _info.num_subcores
NW = NC * NS
B_PER_W = B // NW
CH = 128
NCHUNK = B_PER_W // CH

RL_BS = 4096


RL_GRID = (NROWS + RL_BS - 1) // RL_BS
NPAIR_PAD = RL_GRID * (RL_BS // 2)


def _relayout_body(inT_ref, out_ref):
    x = inT_ref[...]
    out_ref[:, :D] = x[:, :RL_BS // 2].T
    out_ref[:, D:] = x[:, RL_BS // 2:].T


def _relayout(tabT):
    return pl.pallas_call(
        _relayout_body,
        grid=(RL_GRID,),
        in_specs=[pl.BlockSpec((D, RL_BS), lambda i: (0, i))],
        out_specs=pl.BlockSpec((RL_BS // 2, DP), lambda i: (i, 0)),
        out_shape=jax.ShapeDtypeStruct((NPAIR_PAD, DP), jnp.float32),
    )(tabT)


def _make_gather():
    mesh = plsc.VectorSubcoreMesh(core_axis_name="c", subcore_axis_name="s")

    @functools.partial(
        pl.kernel,
        mesh=mesh,
        compiler_params=pltpu.CompilerParams(use_tc_tiling_on_sc=True),
        out_type=(
            jax.ShapeDtypeStruct((B, DP), jnp.float32),
            jax.ShapeDtypeStruct((B, DP), jnp.float32),
        ),
        scratch_types=[
            pltpu.VMEM((NCHUNK, CH), jnp.int32),
            pltpu.VMEM((NCHUNK, CH), jnp.int32),
            pltpu.VMEM((B_PER_W, DP), jnp.float32),
            pltpu.SemaphoreType.DMA,
        ],
    )
    def gather(uidx_hbm, iidx_hbm, user_hbm, item_hbm, u_out, it_out,
               uidx_v, iidx_v, rows, sem):
        wid = lax.axis_index("s") * NC + lax.axis_index("c")
        base = wid * NCHUNK
        rbase = wid * B_PER_W
        pltpu.sync_copy(uidx_hbm.at[pl.ds(base, NCHUNK)], uidx_v)
        pltpu.sync_copy(iidx_hbm.at[pl.ds(base, NCHUNK)], iidx_v)
        copies = []
        for j in range(NCHUNK):
            copies.append(pltpu.async_copy(
                user_hbm.at[uidx_v.at[j]], rows.at[pl.ds(j * CH, CH)], sem))
        for c in copies:
            c.wait()
        pltpu.sync_copy(rows, u_out.at[pl.ds(rbase, B_PER_W)])
        copies = []
        for j in range(NCHUNK):
            copies.append(pltpu.async_copy(
                item_hbm.at[iidx_v.at[j]], rows.at[pl.ds(j * CH, CH)], sem))
        for c in copies:
            c.wait()
        pltpu.sync_copy(rows, it_out.at[pl.ds(rbase, B_PER_W)])

    return gather


_gather = _make_gather()


def _mlp_body(u2_ref, it2_ref, up_ref, ip_ref,
              w1u_ref, w1i_ref, b1_ref, w2_ref, b2_ref, out_ref):
    u = jnp.where(up_ref[...] > 0, u2_ref[:, D:], u2_ref[:, :D])
    it = jnp.where(ip_ref[...] > 0, it2_ref[:, D:], it2_ref[:, :D])
    h = jnp.dot(u, w1u_ref[...], preferred_element_type=jnp.float32)
    h = h + jnp.dot(it, w1i_ref[...], preferred_element_type=jnp.float32)
    h = jnp.maximum(h + b1_ref[0:1, :], 0.0)
    o = jnp.dot(h, w2_ref[...], preferred_element_type=jnp.float32)
    o = o + b2_ref[0:1, :]
    out_ref[...] = jax.nn.sigmoid(o) * (Y_HI - Y_LO) + Y_LO


def _mlp(u2, it2, up, ip, w1u, w1i, b1p, w2p, b2p, bs=2048):
    grid = (B // bs,)
    return pl.pallas_call(
        _mlp_body,
        grid=grid,
        in_specs=[
            pl.BlockSpec((bs, DP), lambda i: (i, 0)),
            pl.BlockSpec((bs, DP), lambda i: (i, 0)),
            pl.BlockSpec((bs, 1), lambda i: (i, 0)),
            pl.BlockSpec((bs, 1), lambda i: (i, 0)),
            pl.BlockSpec((D, N_PAD), lambda i: (0, 0)),
            pl.BlockSpec((D, N_PAD), lambda i: (0, 0)),
            pl.BlockSpec((8, N_PAD), lambda i: (0, 0)),
            pl.BlockSpec((N_PAD, O_PAD), lambda i: (0, 0)),
            pl.BlockSpec((8, O_PAD), lambda i: (0, 0)),
        ],
        out_specs=pl.BlockSpec((bs, O_PAD), lambda i: (i, 0)),
        out_shape=jax.ShapeDtypeStruct((B, O_PAD), jnp.float32),
    )(u2, it2, up, ip, w1u, w1i, b1p, w2p, b2p)


@jax.jit
def kernel(x, user_factors, item_factors0, W1, b1, W2, b2):
    uidx = x[:, 0]
    iidx = x[:, 1]
    half = RL_BS // 2
    uidx2 = ((uidx >> 12) * half + (uidx & (half - 1))).reshape(B // CH, CH)
    up = ((uidx >> 11) & 1).astype(jnp.int32).reshape(B, 1)
    iidx2 = (iidx >> 1).reshape(B // CH, CH)
    ip = (iidx & 1).astype(jnp.int32).reshape(B, 1)

    utab = _relayout(user_factors.T)
    itab = item_factors0.reshape(NPAIR, DP)
    u2, it2 = _gather(uidx2, iidx2, utab, itab)

    w1u = W1[:, :D].T
    w1i = W1[:, D:].T
    w1u = jnp.pad(w1u, ((0, 0), (0, N_PAD - N_ACT)))
    w1i = jnp.pad(w1i, ((0, 0), (0, N_PAD - N_ACT)))
    b1p = jnp.broadcast_to(jnp.pad(b1, (0, N_PAD - N_ACT)), (8, N_PAD))
    w2p = jnp.pad(W2.T, ((0, N_PAD - N_ACT), (0, O_PAD - 5)))
    b2p = jnp.broadcast_to(jnp.pad(b2, (0, O_PAD - 5)), (8, O_PAD))

    out = _mlp(u2, it2, up, ip, w1u, w1i, b1p, w2p, b2p)
    return out[:, :5]


# speedup vs baseline: 1.4357x; 1.4357x over previous
import functools

import jax
import jax.numpy as jnp
from jax import lax
from jax.experimental import pallas as pl
from jax.experimental.pallas import tpu as pltpu
from jax.experimental.pallas import tpu_sc as plsc

B = 16384
D = 64
DP = 128
NROWS = 1000000
N_ACT = 300
N_PAD = 384
O_PAD = 128
Y_LO, Y_HI = 0.0, 5.5

_info = plsc.get_sparse_core_info()
NC, NS = _info.num_cores, _info.num_subcores
NW = NC * NS
B_PER_W = B // NW
CH = 128
NCHUNK = B_PER_W // CH

RL_BS = 4096
RL_Q = RL_BS // 4
RL_GRID = (NROWS + RL_BS - 1) // RL_BS
NQ_PAD = RL_GRID * RL_Q


def _pack_pair(a, b):
    ab = lax.bitcast_convert_type(a.astype(jnp.bfloat16), jnp.uint16)
    bb = lax.bitcast_convert_type(b.astype(jnp.bfloat16), jnp.uint16)
    w = (ab.astype(jnp.uint32) << 16) | bb.astype(jnp.uint32)
    return lax.bitcast_convert_type(w, jnp.float32)


def _relayout_body(inT_ref, out_ref):
    x = inT_ref[...]
    q0 = x[:, 0 * RL_Q:1 * RL_Q].T
    q1 = x[:, 1 * RL_Q:2 * RL_Q].T
    q2 = x[:, 2 * RL_Q:3 * RL_Q].T
    q3 = x[:, 3 * RL_Q:4 * RL_Q].T
    out_ref[:, :D] = _pack_pair(q0, q1)
    out_ref[:, D:] = _pack_pair(q2, q3)


def _relayout(tabT):
    return pl.pallas_call(
        _relayout_body,
        grid=(RL_GRID,),
        in_specs=[pl.BlockSpec((D, RL_BS), lambda i: (0, i))],
        out_specs=pl.BlockSpec((RL_Q, DP), lambda i: (i, 0)),
        out_shape=jax.ShapeDtypeStruct((NQ_PAD, DP), jnp.float32),
    )(tabT)


def _make_gather():
    mesh = plsc.VectorSubcoreMesh(core_axis_name="c", subcore_axis_name="s")

    @functools.partial(
        pl.kernel,
        mesh=mesh,
        compiler_params=pltpu.CompilerParams(use_tc_tiling_on_sc=True),
        out_type=(
            jax.ShapeDtypeStruct((B, DP), jnp.float32),
            jax.ShapeDtypeStruct((B, DP), jnp.float32),
        ),
        scratch_types=[
            pltpu.VMEM((NCHUNK, CH), jnp.int32),
            pltpu.VMEM((NCHUNK, CH), jnp.int32),
            pltpu.VMEM((B_PER_W, DP), jnp.float32),
            pltpu.SemaphoreType.DMA,
        ],
    )
    def gather(uidx_hbm, iidx_hbm, user_hbm, item_hbm, u_out, it_out,
               uidx_v, iidx_v, rows, sem):
        wid = lax.axis_index("s") * NC + lax.axis_index("c")
        base = wid * NCHUNK
        rbase = wid * B_PER_W
        pltpu.sync_copy(uidx_hbm.at[pl.ds(base, NCHUNK)], uidx_v)
        pltpu.sync_copy(iidx_hbm.at[pl.ds(base, NCHUNK)], iidx_v)
        copies = []
        for j in range(NCHUNK):
            copies.append(pltpu.async_copy(
                user_hbm.at[uidx_v.at[j]], rows.at[pl.ds(j * CH, CH)], sem))
        for c in copies:
            c.wait()
        pltpu.sync_copy(rows, u_out.at[pl.ds(rbase, B_PER_W)])
        copies = []
        for j in range(NCHUNK):
            copies.append(pltpu.async_copy(
                item_hbm.at[iidx_v.at[j]], rows.at[pl.ds(j * CH, CH)], sem))
        for c in copies:
            c.wait()
        pltpu.sync_copy(rows, it_out.at[pl.ds(rbase, B_PER_W)])

    return gather


_gather = _make_gather()


def _unpack_select(u2, half, hi):
    w = jnp.where(half > 0, u2[:, D:], u2[:, :D])
    bits = lax.bitcast_convert_type(w, jnp.uint32)
    b16 = jnp.where(hi > 0, bits & jnp.uint32(0xFFFF0000), bits << 16)
    return lax.bitcast_convert_type(b16, jnp.float32)


def _mlp_body(u2_ref, it2_ref, us_ref, is_ref,
              w1u_ref, w1i_ref, b1_ref, w2_ref, b2_ref, out_ref):
    us = us_ref[...]
    isel = is_ref[...]
    u = _unpack_select(u2_ref[...], us & 2, 1 - (us & 1))
    it = _unpack_select(it2_ref[...], isel & 2, 1 - (isel & 1))
    h = jnp.dot(u, w1u_ref[...], preferred_element_type=jnp.float32)
    h = h + jnp.dot(it, w1i_ref[...], preferred_element_type=jnp.float32)
    h = jnp.maximum(h + b1_ref[0:1, :], 0.0)
    o = jnp.dot(h, w2_ref[...], preferred_element_type=jnp.float32)
    o = o + b2_ref[0:1, :]
    out_ref[...] = jax.nn.sigmoid(o) * (Y_HI - Y_LO) + Y_LO


def _mlp(u2, it2, us, isel, w1u, w1i, b1p, w2p, b2p, bs=2048):
    grid = (B // bs,)
    return pl.pallas_call(
        _mlp_body,
        grid=grid,
        in_specs=[
            pl.BlockSpec((bs, DP), lambda i: (i, 0)),
            pl.BlockSpec((bs, DP), lambda i: (i, 0)),
            pl.BlockSpec((bs, 1), lambda i: (i, 0)),
            pl.BlockSpec((bs, 1), lambda i: (i, 0)),
            pl.BlockSpec((D, N_PAD), lambda i: (0, 0)),
            pl.BlockSpec((D, N_PAD), lambda i: (0, 0)),
            pl.BlockSpec((8, N_PAD), lambda i: (0, 0)),
            pl.BlockSpec((N_PAD, O_PAD), lambda i: (0, 0)),
            pl.BlockSpec((8, O_PAD), lambda i: (0, 0)),
        ],
        out_specs=pl.BlockSpec((bs, O_PAD), lambda i: (i, 0)),
        out_shape=jax.ShapeDtypeStruct((B, O_PAD), jnp.float32),
    )(u2, it2, us, isel, w1u, w1i, b1p, w2p, b2p)


@jax.jit
def kernel(x, user_factors, item_factors0, W1, b1, W2, b2):
    uidx = x[:, 0]
    iidx = x[:, 1]
    uidx2 = ((uidx >> 12) * RL_Q + (uidx & (RL_Q - 1))).reshape(B // CH, CH)
    iidx2 = ((iidx >> 12) * RL_Q + (iidx & (RL_Q - 1))).reshape(B // CH, CH)
    us = ((uidx >> 10) & 3).astype(jnp.int32).reshape(B, 1)
    isel = ((iidx >> 10) & 3).astype(jnp.int32).reshape(B, 1)

    utab = _relayout(user_factors.T)
    itab = _relayout(item_factors0.T)
    u2, it2 = _gather(uidx2, iidx2, utab, itab)

    w1u = W1[:, :D].T
    w1i = W1[:, D:].T
    w1u = jnp.pad(w1u, ((0, 0), (0, N_PAD - N_ACT)))
    w1i = jnp.pad(w1i, ((0, 0), (0, N_PAD - N_ACT)))
    b1p = jnp.broadcast_to(jnp.pad(b1, (0, N_PAD - N_ACT)), (8, N_PAD))
    w2p = jnp.pad(W2.T, ((0, N_PAD - N_ACT), (0, O_PAD - 5)))
    b2p = jnp.broadcast_to(jnp.pad(b2, (0, O_PAD - 5)), (8, O_PAD))

    out = _mlp(u2, it2, us, isel, w1u, w1i, b1p, w2p, b2p)
    return out[:, :5]


# speedup vs baseline: 1.5070x; 1.0497x over previous
import functools

import jax
import jax.numpy as jnp
from jax import lax
from jax.experimental import pallas as pl
from jax.experimental.pallas import tpu as pltpu
from jax.experimental.pallas import tpu_sc as plsc

B = 16384
D = 64
DP = 128
NROWS = 1000000
N_ACT = 300
N_PAD = 384
O_PAD = 8
Y_LO, Y_HI = 0.0, 5.5

_info = plsc.get_sparse_core_info()
NC, NS = _info.num_cores, _info.num_subcores
NW = NC * NS
B_PER_W = B // NW
CH = 128
NCHUNK = B_PER_W // CH

RL_BS = 4096
RL_Q = RL_BS // 4
RL_GRID = (NROWS + RL_BS - 1) // RL_BS
NQ_PAD = RL_GRID * RL_Q


def _pack_pair(a, b):
    ab = lax.bitcast_convert_type(a.astype(jnp.bfloat16), jnp.uint16)
    bb = lax.bitcast_convert_type(b.astype(jnp.bfloat16), jnp.uint16)
    w = (ab.astype(jnp.uint32) << 16) | bb.astype(jnp.uint32)
    return lax.bitcast_convert_type(w, jnp.float32)


def _relayout_body(inT_ref, out_ref):
    x = inT_ref[...]
    q0 = x[:, 0 * RL_Q:1 * RL_Q].T
    q1 = x[:, 1 * RL_Q:2 * RL_Q].T
    q2 = x[:, 2 * RL_Q:3 * RL_Q].T
    q3 = x[:, 3 * RL_Q:4 * RL_Q].T
    out_ref[:, :D] = _pack_pair(q0, q1)
    out_ref[:, D:] = _pack_pair(q2, q3)


def _relayout(tabT):
    return pl.pallas_call(
        _relayout_body,
        grid=(RL_GRID,),
        in_specs=[pl.BlockSpec((D, RL_BS), lambda i: (0, i))],
        out_specs=pl.BlockSpec((RL_Q, DP), lambda i: (i, 0)),
        out_shape=jax.ShapeDtypeStruct((NQ_PAD, DP), jnp.float32),
    )(tabT)


def _make_gather():
    mesh = plsc.VectorSubcoreMesh(core_axis_name="c", subcore_axis_name="s")

    @functools.partial(
        pl.kernel,
        mesh=mesh,
        compiler_params=pltpu.CompilerParams(use_tc_tiling_on_sc=True),
        out_type=(
            jax.ShapeDtypeStruct((B, DP), jnp.float32),
            jax.ShapeDtypeStruct((B, DP), jnp.float32),
        ),
        scratch_types=[
            pltpu.VMEM((NCHUNK, CH), jnp.int32),
            pltpu.VMEM((NCHUNK, CH), jnp.int32),
            pltpu.VMEM((B_PER_W, DP), jnp.float32),
            pltpu.SemaphoreType.DMA,
        ],
    )
    def gather(uidx_hbm, iidx_hbm, user_hbm, item_hbm, u_out, it_out,
               uidx_v, iidx_v, rows, sem):
        wid = lax.axis_index("s") * NC + lax.axis_index("c")
        base = wid * NCHUNK
        rbase = wid * B_PER_W
        pltpu.sync_copy(uidx_hbm.at[pl.ds(base, NCHUNK)], uidx_v)
        pltpu.sync_copy(iidx_hbm.at[pl.ds(base, NCHUNK)], iidx_v)
        copies = []
        for j in range(NCHUNK):
            copies.append(pltpu.async_copy(
                user_hbm.at[uidx_v.at[j]], rows.at[pl.ds(j * CH, CH)], sem))
        for c in copies:
            c.wait()
        pltpu.sync_copy(rows, u_out.at[pl.ds(rbase, B_PER_W)])
        copies = []
        for j in range(NCHUNK):
            copies.append(pltpu.async_copy(
                item_hbm.at[iidx_v.at[j]], rows.at[pl.ds(j * CH, CH)], sem))
        for c in copies:
            c.wait()
        pltpu.sync_copy(rows, it_out.at[pl.ds(rbase, B_PER_W)])

    return gather


_gather = _make_gather()


def _unpack_select(u2, half, hi):
    w = jnp.where(half > 0, u2[:, D:], u2[:, :D])
    bits = lax.bitcast_convert_type(w, jnp.uint32)
    b16 = jnp.where(hi > 0, bits & jnp.uint32(0xFFFF0000), bits << 16)
    return lax.bitcast_convert_type(b16, jnp.float32)


def _mlp_body(u2_ref, it2_ref, us_ref, is_ref,
              w1u_ref, w1i_ref, b1_ref, w2_ref, b2_ref, out_ref):
    us = us_ref[:, 0:1]
    isel = is_ref[:, 0:1]
    u = _unpack_select(u2_ref[...], us & 2, 1 - (us & 1))
    it = _unpack_select(it2_ref[...], isel & 2, 1 - (isel & 1))
    h = jnp.dot(u, w1u_ref[...], preferred_element_type=jnp.float32)
    h = h + jnp.dot(it, w1i_ref[...], preferred_element_type=jnp.float32)
    h = jnp.maximum(h + b1_ref[0:1, :], 0.0)
    o = jnp.dot(h, w2_ref[...], preferred_element_type=jnp.float32)
    o = o + b2_ref[0:1, :]
    out_ref[...] = jax.nn.sigmoid(o) * (Y_HI - Y_LO) + Y_LO


def _mlp(u2, it2, us, isel, w1u, w1i, b1p, w2p, b2p, bs=2048):
    grid = (B // bs,)
    return pl.pallas_call(
        _mlp_body,
        grid=grid,
        in_specs=[
            pl.BlockSpec((bs, DP), lambda i: (i, 0)),
            pl.BlockSpec((bs, DP), lambda i: (i, 0)),
            pl.BlockSpec((bs, 8), lambda i: (i, 0)),
            pl.BlockSpec((bs, 8), lambda i: (i, 0)),
            pl.BlockSpec((D, N_PAD), lambda i: (0, 0)),
            pl.BlockSpec((D, N_PAD), lambda i: (0, 0)),
            pl.BlockSpec((8, N_PAD), lambda i: (0, 0)),
            pl.BlockSpec((N_PAD, O_PAD), lambda i: (0, 0)),
            pl.BlockSpec((8, O_PAD), lambda i: (0, 0)),
        ],
        out_specs=pl.BlockSpec((bs, O_PAD), lambda i: (i, 0)),
        out_shape=jax.ShapeDtypeStruct((B, O_PAD), jnp.float32),
    )(u2, it2, us, isel, w1u, w1i, b1p, w2p, b2p)


@jax.jit
def kernel(x, user_factors, item_factors0, W1, b1, W2, b2):
    uidx = x[:, 0]
    iidx = x[:, 1]
    uidx2 = ((uidx >> 12) * RL_Q + (uidx & (RL_Q - 1))).reshape(B // CH, CH)
    iidx2 = ((iidx >> 12) * RL_Q + (iidx & (RL_Q - 1))).reshape(B // CH, CH)
    us = jnp.broadcast_to(
        ((uidx >> 10) & 3).astype(jnp.int32).reshape(B, 1), (B, 8))
    isel = jnp.broadcast_to(
        ((iidx >> 10) & 3).astype(jnp.int32).reshape(B, 1), (B, 8))

    utab = _relayout(user_factors.T)
    itab = _relayout(item_factors0.T)
    u2, it2 = _gather(uidx2, iidx2, utab, itab)

    w1u = W1[:, :D].T
    w1i = W1[:, D:].T
    w1u = jnp.pad(w1u, ((0, 0), (0, N_PAD - N_ACT)))
    w1i = jnp.pad(w1i, ((0, 0), (0, N_PAD - N_ACT)))
    b1p = jnp.broadcast_to(jnp.pad(b1, (0, N_PAD - N_ACT)), (8, N_PAD))
    w2p = jnp.pad(W2.T, ((0, N_PAD - N_ACT), (0, O_PAD - 5)))
    b2p = jnp.broadcast_to(jnp.pad(b2, (0, O_PAD - 5)), (8, O_PAD))

    out = _mlp(u2, it2, us, isel, w1u, w1i, b1p, w2p, b2p)
    return out[:, :5]


# speedup vs baseline: 1.9059x; 1.2647x over previous
import functools

import jax
import jax.numpy as jnp
from jax import lax
from jax.experimental import pallas as pl
from jax.experimental.pallas import tpu as pltpu
from jax.experimental.pallas import tpu_sc as plsc

B = 16384
D = 64
DP = 128
NROWS = 1000000
N_ACT = 300
N_PAD = 384
O_PAD = 8
Y_LO, Y_HI = 0.0, 5.5

_info = plsc.get_sparse_core_info()
NC, NS = _info.num_cores, _info.num_subcores
NW = NC * NS
B_PER_W = B // NW
CH = 128
NCHUNK = B_PER_W // CH

RL_BS = 8192
RL_LOG = 13
RL_Q = RL_BS // 4
RL_GRID = (NROWS + RL_BS - 1) // RL_BS
NQ_PAD = RL_GRID * RL_Q


def _pack_pair(a, b):
    ab = lax.bitcast_convert_type(a.astype(jnp.bfloat16), jnp.uint16)
    bb = lax.bitcast_convert_type(b.astype(jnp.bfloat16), jnp.uint16)
    w = (ab.astype(jnp.uint32) << 16) | bb.astype(jnp.uint32)
    return lax.bitcast_convert_type(w, jnp.float32)


def _relayout_body(inT_ref, out_ref):
    x = inT_ref[...]
    q0 = x[:, 0 * RL_Q:1 * RL_Q].T
    q1 = x[:, 1 * RL_Q:2 * RL_Q].T
    q2 = x[:, 2 * RL_Q:3 * RL_Q].T
    q3 = x[:, 3 * RL_Q:4 * RL_Q].T
    out_ref[:, :D] = _pack_pair(q0, q1)
    out_ref[:, D:] = _pack_pair(q2, q3)


def _relayout(tabT):
    return pl.pallas_call(
        _relayout_body,
        grid=(RL_GRID,),
        in_specs=[pl.BlockSpec((D, RL_BS), lambda i: (0, i))],
        out_specs=pl.BlockSpec((RL_Q, DP), lambda i: (i, 0)),
        out_shape=jax.ShapeDtypeStruct((NQ_PAD, DP), jnp.float32),
    )(tabT)


def _make_gather():
    mesh = plsc.VectorSubcoreMesh(core_axis_name="c", subcore_axis_name="s")

    @functools.partial(
        pl.kernel,
        mesh=mesh,
        compiler_params=pltpu.CompilerParams(use_tc_tiling_on_sc=True),
        out_type=(
            jax.ShapeDtypeStruct((B, DP), jnp.float32),
            jax.ShapeDtypeStruct((B, DP), jnp.float32),
        ),
        scratch_types=[
            pltpu.VMEM((NCHUNK, CH), jnp.int32),
            pltpu.VMEM((NCHUNK, CH), jnp.int32),
            pltpu.VMEM((B_PER_W, DP), jnp.float32),
            pltpu.SemaphoreType.DMA,
        ],
    )
    def gather(uidx_hbm, iidx_hbm, user_hbm, item_hbm, u_out, it_out,
               uidx_v, iidx_v, rows, sem):
        wid = lax.axis_index("s") * NC + lax.axis_index("c")
        base = wid * NCHUNK
        rbase = wid * B_PER_W
        pltpu.sync_copy(uidx_hbm.at[pl.ds(base, NCHUNK)], uidx_v)
        pltpu.sync_copy(iidx_hbm.at[pl.ds(base, NCHUNK)], iidx_v)
        copies = []
        for j in range(NCHUNK):
            copies.append(pltpu.async_copy(
                user_hbm.at[uidx_v.at[j]], rows.at[pl.ds(j * CH, CH)], sem))
        for c in copies:
            c.wait()
        pltpu.sync_copy(rows, u_out.at[pl.ds(rbase, B_PER_W)])
        copies = []
        for j in range(NCHUNK):
            copies.append(pltpu.async_copy(
                item_hbm.at[iidx_v.at[j]], rows.at[pl.ds(j * CH, CH)], sem))
        for c in copies:
            c.wait()
        pltpu.sync_copy(rows, it_out.at[pl.ds(rbase, B_PER_W)])

    return gather


_gather = _make_gather()


def _unpack_select(u2, half, hi):
    w = jnp.where(half > 0, u2[:, D:], u2[:, :D])
    bits = lax.bitcast_convert_type(w, jnp.uint32)
    b16 = jnp.where(hi > 0, bits & jnp.uint32(0xFFFF0000), bits << 16)
    return lax.bitcast_convert_type(b16, jnp.float32)


def _mlp_body(u2_ref, it2_ref, us_ref, is_ref,
              w1u_ref, w1i_ref, b1_ref, w2_ref, b2_ref, out_ref):
    us = us_ref[:, 0:1]
    isel = is_ref[:, 0:1]
    u = _unpack_select(u2_ref[...], us & 2, 1 - (us & 1))
    it = _unpack_select(it2_ref[...], isel & 2, 1 - (isel & 1))
    h = jnp.dot(u, w1u_ref[...], preferred_element_type=jnp.float32)
    h = h + jnp.dot(it, w1i_ref[...], preferred_element_type=jnp.float32)
    h = jnp.maximum(h + b1_ref[0:1, :], 0.0)
    o = jnp.dot(h, w2_ref[...], preferred_element_type=jnp.float32)
    o = o + b2_ref[0:1, :]
    out_ref[...] = jax.nn.sigmoid(o) * (Y_HI - Y_LO) + Y_LO


def _mlp(u2, it2, us, isel, w1u, w1i, b1p, w2p, b2p, bs=2048):
    grid = (B // bs,)
    return pl.pallas_call(
        _mlp_body,
        grid=grid,
        in_specs=[
            pl.BlockSpec((bs, DP), lambda i: (i, 0)),
            pl.BlockSpec((bs, DP), lambda i: (i, 0)),
            pl.BlockSpec((bs, 8), lambda i: (i, 0)),
            pl.BlockSpec((bs, 8), lambda i: (i, 0)),
            pl.BlockSpec((D, N_PAD), lambda i: (0, 0)),
            pl.BlockSpec((D, N_PAD), lambda i: (0, 0)),
            pl.BlockSpec((8, N_PAD), lambda i: (0, 0)),
            pl.BlockSpec((N_PAD, O_PAD), lambda i: (0, 0)),
            pl.BlockSpec((8, O_PAD), lambda i: (0, 0)),
        ],
        out_specs=pl.BlockSpec((bs, O_PAD), lambda i: (i, 0)),
        out_shape=jax.ShapeDtypeStruct((B, O_PAD), jnp.float32),
    )(u2, it2, us, isel, w1u, w1i, b1p, w2p, b2p)


@jax.jit
def kernel(x, user_factors, item_factors0, W1, b1, W2, b2):
    uidx = x[:, 0]
    iidx = x[:, 1]
    uidx2 = ((uidx >> RL_LOG) * RL_Q + (uidx & (RL_Q - 1))).reshape(B // CH, CH)
    iidx2 = ((iidx >> RL_LOG) * RL_Q + (iidx & (RL_Q - 1))).reshape(B // CH, CH)
    us = jnp.broadcast_to(
        ((uidx >> (RL_LOG - 2)) & 3).astype(jnp.int32).reshape(B, 1), (B, 8))
    isel = jnp.broadcast_to(
        ((iidx >> (RL_LOG - 2)) & 3).astype(jnp.int32).reshape(B, 1), (B, 8))

    utab = _relayout(user_factors.T)
    itab = _relayout(item_factors0.T)
    u2, it2 = _gather(uidx2, iidx2, utab, itab)

    w1u = W1[:, :D].T
    w1i = W1[:, D:].T
    w1u = jnp.pad(w1u, ((0, 0), (0, N_PAD - N_ACT)))
    w1i = jnp.pad(w1i, ((0, 0), (0, N_PAD - N_ACT)))
    b1p = jnp.broadcast_to(jnp.pad(b1, (0, N_PAD - N_ACT)), (8, N_PAD))
    w2p = jnp.pad(W2.T, ((0, N_PAD - N_ACT), (0, O_PAD - 5)))
    b2p = jnp.broadcast_to(jnp.pad(b2, (0, O_PAD - 5)), (8, O_PAD))

    out = _mlp(u2, it2, us, isel, w1u, w1i, b1p, w2p, b2p)
    return out[:, :5]


# speedup vs baseline: 2.2618x; 1.1867x over previous
import functools

import jax
import jax.numpy as jnp
from jax import lax
from jax.experimental import pallas as pl
from jax.experimental.pallas import tpu as pltpu
from jax.experimental.pallas import tpu_sc as plsc

B = 16384
D = 64
DP = 128
NROWS = 1000000
N_ACT = 300
N_PAD = 384
O_PAD = 8
Y_LO, Y_HI = 0.0, 5.5

_info = plsc.get_sparse_core_info()
NC, NS = _info.num_cores, _info.num_subcores
NW = NC * NS
B_PER_W = B // NW
CH = 128
NCHUNK = B_PER_W // CH

RL_BS = 16384
RL_LOG = 14
RL_Q = RL_BS // 4
RL_GRID = (NROWS + RL_BS - 1) // RL_BS
NQ_PAD = RL_GRID * RL_Q


def _pack_pair(a, b):
    ab = lax.bitcast_convert_type(a.astype(jnp.bfloat16), jnp.uint16)
    bb = lax.bitcast_convert_type(b.astype(jnp.bfloat16), jnp.uint16)
    w = (ab.astype(jnp.uint32) << 16) | bb.astype(jnp.uint32)
    return lax.bitcast_convert_type(w, jnp.float32)


def _relayout_body(inT_ref, out_ref):
    x = inT_ref[...]
    q0 = x[:, 0 * RL_Q:1 * RL_Q].T
    q1 = x[:, 1 * RL_Q:2 * RL_Q].T
    q2 = x[:, 2 * RL_Q:3 * RL_Q].T
    q3 = x[:, 3 * RL_Q:4 * RL_Q].T
    out_ref[:, :D] = _pack_pair(q0, q1)
    out_ref[:, D:] = _pack_pair(q2, q3)


def _relayout(tabT):
    return pl.pallas_call(
        _relayout_body,
        grid=(RL_GRID,),
        in_specs=[pl.BlockSpec((D, RL_BS), lambda i: (0, i))],
        out_specs=pl.BlockSpec((RL_Q, DP), lambda i: (i, 0)),
        out_shape=jax.ShapeDtypeStruct((NQ_PAD, DP), jnp.float32),
    )(tabT)


def _make_gather():
    mesh = plsc.VectorSubcoreMesh(core_axis_name="c", subcore_axis_name="s")

    @functools.partial(
        pl.kernel,
        mesh=mesh,
        compiler_params=pltpu.CompilerParams(use_tc_tiling_on_sc=True),
        out_type=(
            jax.ShapeDtypeStruct((B, DP), jnp.float32),
            jax.ShapeDtypeStruct((B, DP), jnp.float32),
        ),
        scratch_types=[
            pltpu.VMEM((NCHUNK, CH), jnp.int32),
            pltpu.VMEM((NCHUNK, CH), jnp.int32),
            pltpu.VMEM((B_PER_W, DP), jnp.float32),
            pltpu.SemaphoreType.DMA,
        ],
    )
    def gather(uidx_hbm, iidx_hbm, user_hbm, item_hbm, u_out, it_out,
               uidx_v, iidx_v, rows, sem):
        wid = lax.axis_index("s") * NC + lax.axis_index("c")
        base = wid * NCHUNK
        rbase = wid * B_PER_W
        pltpu.sync_copy(uidx_hbm.at[pl.ds(base, NCHUNK)], uidx_v)
        pltpu.sync_copy(iidx_hbm.at[pl.ds(base, NCHUNK)], iidx_v)
        copies = []
        for j in range(NCHUNK):
            copies.append(pltpu.async_copy(
                user_hbm.at[uidx_v.at[j]], rows.at[pl.ds(j * CH, CH)], sem))
        for c in copies:
            c.wait()
        pltpu.sync_copy(rows, u_out.at[pl.ds(rbase, B_PER_W)])
        copies = []
        for j in range(NCHUNK):
            copies.append(pltpu.async_copy(
                item_hbm.at[iidx_v.at[j]], rows.at[pl.ds(j * CH, CH)], sem))
        for c in copies:
            c.wait()
        pltpu.sync_copy(rows, it_out.at[pl.ds(rbase, B_PER_W)])

    return gather


_gather = _make_gather()


def _unpack_select(u2, half, hi):
    w = jnp.where(half > 0, u2[:, D:], u2[:, :D])
    bits = lax.bitcast_convert_type(w, jnp.uint32)
    b16 = jnp.where(hi > 0, bits & jnp.uint32(0xFFFF0000), bits << 16)
    return lax.bitcast_convert_type(b16, jnp.float32)


def _mlp_body(u2_ref, it2_ref, us_ref, is_ref,
              w1u_ref, w1i_ref, b1_ref, w2_ref, b2_ref, out_ref):
    us = us_ref[:, 0:1]
    isel = is_ref[:, 0:1]
    u = _unpack_select(u2_ref[...], us & 2, 1 - (us & 1))
    it = _unpack_select(it2_ref[...], isel & 2, 1 - (isel & 1))
    h = jnp.dot(u, w1u_ref[...], preferred_element_type=jnp.float32)
    h = h + jnp.dot(it, w1i_ref[...], preferred_element_type=jnp.float32)
    h = jnp.maximum(h + b1_ref[0:1, :], 0.0)
    o = jnp.dot(h, w2_ref[...], preferred_element_type=jnp.float32)
    o = o + b2_ref[0:1, :]
    out_ref[...] = jax.nn.sigmoid(o) * (Y_HI - Y_LO) + Y_LO


def _mlp(u2, it2, us, isel, w1u, w1i, b1p, w2p, b2p, bs=2048):
    grid = (B // bs,)
    return pl.pallas_call(
        _mlp_body,
        grid=grid,
        in_specs=[
            pl.BlockSpec((bs, DP), lambda i: (i, 0)),
            pl.BlockSpec((bs, DP), lambda i: (i, 0)),
            pl.BlockSpec((bs, 8), lambda i: (i, 0)),
            pl.BlockSpec((bs, 8), lambda i: (i, 0)),
            pl.BlockSpec((D, N_PAD), lambda i: (0, 0)),
            pl.BlockSpec((D, N_PAD), lambda i: (0, 0)),
            pl.BlockSpec((8, N_PAD), lambda i: (0, 0)),
            pl.BlockSpec((N_PAD, O_PAD), lambda i: (0, 0)),
            pl.BlockSpec((8, O_PAD), lambda i: (0, 0)),
        ],
        out_specs=pl.BlockSpec((bs, O_PAD), lambda i: (i, 0)),
        out_shape=jax.ShapeDtypeStruct((B, O_PAD), jnp.float32),
    )(u2, it2, us, isel, w1u, w1i, b1p, w2p, b2p)


@jax.jit
def kernel(x, user_factors, item_factors0, W1, b1, W2, b2):
    uidx = x[:, 0]
    iidx = x[:, 1]
    uidx2 = ((uidx >> RL_LOG) * RL_Q + (uidx & (RL_Q - 1))).reshape(B // CH, CH)
    iidx2 = ((iidx >> RL_LOG) * RL_Q + (iidx & (RL_Q - 1))).reshape(B // CH, CH)
    us = jnp.broadcast_to(
        ((uidx >> (RL_LOG - 2)) & 3).astype(jnp.int32).reshape(B, 1), (B, 8))
    isel = jnp.broadcast_to(
        ((iidx >> (RL_LOG - 2)) & 3).astype(jnp.int32).reshape(B, 1), (B, 8))

    utab = _relayout(user_factors.T)
    itab = _relayout(item_factors0.T)
    u2, it2 = _gather(uidx2, iidx2, utab, itab)

    w1u = W1[:, :D].T
    w1i = W1[:, D:].T
    w1u = jnp.pad(w1u, ((0, 0), (0, N_PAD - N_ACT)))
    w1i = jnp.pad(w1i, ((0, 0), (0, N_PAD - N_ACT)))
    b1p = jnp.broadcast_to(jnp.pad(b1, (0, N_PAD - N_ACT)), (8, N_PAD))
    w2p = jnp.pad(W2.T, ((0, N_PAD - N_ACT), (0, O_PAD - 5)))
    b2p = jnp.broadcast_to(jnp.pad(b2, (0, O_PAD - 5)), (8, O_PAD))

    out = _mlp(u2, it2, us, isel, w1u, w1i, b1p, w2p, b2p)
    return out[:, :5]


# speedup vs baseline: 2.5036x; 1.1069x over previous
import functools

import jax
import jax.numpy as jnp
from jax import lax
from jax.experimental import pallas as pl
from jax.experimental.pallas import tpu as pltpu
from jax.experimental.pallas import tpu_sc as plsc

B = 16384
D = 64
DP = 128
NROWS = 1000000
N_ACT = 300
N_PAD = 384
O_PAD = 8
Y_LO, Y_HI = 0.0, 5.5

_info = plsc.get_sparse_core_info()
NC, NS = _info.num_cores, _info.num_subcores
NW = NC * NS
B_PER_W = B // NW
CH = 128
NCHUNK = B_PER_W // CH

RL_BS = 32768
RL_LOG = 15
RL_Q = RL_BS // 4
RL_GRID = (NROWS + RL_BS - 1) // RL_BS
NQ_PAD = RL_GRID * RL_Q


def _pack_pair(a, b):
    ab = lax.bitcast_convert_type(a.astype(jnp.bfloat16), jnp.uint16)
    bb = lax.bitcast_convert_type(b.astype(jnp.bfloat16), jnp.uint16)
    w = (ab.astype(jnp.uint32) << 16) | bb.astype(jnp.uint32)
    return lax.bitcast_convert_type(w, jnp.float32)


def _relayout_body(inT_ref, out_ref):
    x = inT_ref[...]
    q0 = x[:, 0 * RL_Q:1 * RL_Q].T
    q1 = x[:, 1 * RL_Q:2 * RL_Q].T
    q2 = x[:, 2 * RL_Q:3 * RL_Q].T
    q3 = x[:, 3 * RL_Q:4 * RL_Q].T
    out_ref[:, :D] = _pack_pair(q0, q1)
    out_ref[:, D:] = _pack_pair(q2, q3)


def _relayout(tabT):
    return pl.pallas_call(
        _relayout_body,
        grid=(RL_GRID,),
        in_specs=[pl.BlockSpec((D, RL_BS), lambda i: (0, i))],
        out_specs=pl.BlockSpec((RL_Q, DP), lambda i: (i, 0)),
        out_shape=jax.ShapeDtypeStruct((NQ_PAD, DP), jnp.float32),
    )(tabT)


def _make_gather():
    mesh = plsc.VectorSubcoreMesh(core_axis_name="c", subcore_axis_name="s")

    @functools.partial(
        pl.kernel,
        mesh=mesh,
        compiler_params=pltpu.CompilerParams(use_tc_tiling_on_sc=True),
        out_type=(
            jax.ShapeDtypeStruct((B, DP), jnp.float32),
            jax.ShapeDtypeStruct((B, DP), jnp.float32),
        ),
        scratch_types=[
            pltpu.VMEM((NCHUNK, CH), jnp.int32),
            pltpu.VMEM((NCHUNK, CH), jnp.int32),
            pltpu.VMEM((B_PER_W, DP), jnp.float32),
            pltpu.SemaphoreType.DMA,
        ],
    )
    def gather(uidx_hbm, iidx_hbm, user_hbm, item_hbm, u_out, it_out,
               uidx_v, iidx_v, rows, sem):
        wid = lax.axis_index("s") * NC + lax.axis_index("c")
        base = wid * NCHUNK
        rbase = wid * B_PER_W
        pltpu.sync_copy(uidx_hbm.at[pl.ds(base, NCHUNK)], uidx_v)
        pltpu.sync_copy(iidx_hbm.at[pl.ds(base, NCHUNK)], iidx_v)
        copies = []
        for j in range(NCHUNK):
            copies.append(pltpu.async_copy(
                user_hbm.at[uidx_v.at[j]], rows.at[pl.ds(j * CH, CH)], sem))
        for c in copies:
            c.wait()
        pltpu.sync_copy(rows, u_out.at[pl.ds(rbase, B_PER_W)])
        copies = []
        for j in range(NCHUNK):
            copies.append(pltpu.async_copy(
                item_hbm.at[iidx_v.at[j]], rows.at[pl.ds(j * CH, CH)], sem))
        for c in copies:
            c.wait()
        pltpu.sync_copy(rows, it_out.at[pl.ds(rbase, B_PER_W)])

    return gather


_gather = _make_gather()


def _unpack_select(u2, half, hi):
    w = jnp.where(half > 0, u2[:, D:], u2[:, :D])
    bits = lax.bitcast_convert_type(w, jnp.uint32)
    b16 = jnp.where(hi > 0, bits & jnp.uint32(0xFFFF0000), bits << 16)
    return lax.bitcast_convert_type(b16, jnp.float32)


def _mlp_body(u2_ref, it2_ref, us_ref, is_ref,
              w1u_ref, w1i_ref, b1_ref, w2_ref, b2_ref, out_ref):
    us = us_ref[:, 0:1]
    isel = is_ref[:, 0:1]
    u = _unpack_select(u2_ref[...], us & 2, 1 - (us & 1))
    it = _unpack_select(it2_ref[...], isel & 2, 1 - (isel & 1))
    h = jnp.dot(u, w1u_ref[...], preferred_element_type=jnp.float32)
    h = h + jnp.dot(it, w1i_ref[...], preferred_element_type=jnp.float32)
    h = jnp.maximum(h + b1_ref[0:1, :], 0.0)
    o = jnp.dot(h, w2_ref[...], preferred_element_type=jnp.float32)
    o = o + b2_ref[0:1, :]
    out_ref[...] = jax.nn.sigmoid(o) * (Y_HI - Y_LO) + Y_LO


def _mlp(u2, it2, us, isel, w1u, w1i, b1p, w2p, b2p, bs=2048):
    grid = (B // bs,)
    return pl.pallas_call(
        _mlp_body,
        grid=grid,
        in_specs=[
            pl.BlockSpec((bs, DP), lambda i: (i, 0)),
            pl.BlockSpec((bs, DP), lambda i: (i, 0)),
            pl.BlockSpec((bs, 8), lambda i: (i, 0)),
            pl.BlockSpec((bs, 8), lambda i: (i, 0)),
            pl.BlockSpec((D, N_PAD), lambda i: (0, 0)),
            pl.BlockSpec((D, N_PAD), lambda i: (0, 0)),
            pl.BlockSpec((8, N_PAD), lambda i: (0, 0)),
            pl.BlockSpec((N_PAD, O_PAD), lambda i: (0, 0)),
            pl.BlockSpec((8, O_PAD), lambda i: (0, 0)),
        ],
        out_specs=pl.BlockSpec((bs, O_PAD), lambda i: (i, 0)),
        out_shape=jax.ShapeDtypeStruct((B, O_PAD), jnp.float32),
    )(u2, it2, us, isel, w1u, w1i, b1p, w2p, b2p)


@jax.jit
def kernel(x, user_factors, item_factors0, W1, b1, W2, b2):
    uidx = x[:, 0]
    iidx = x[:, 1]
    uidx2 = ((uidx >> RL_LOG) * RL_Q + (uidx & (RL_Q - 1))).reshape(B // CH, CH)
    iidx2 = ((iidx >> RL_LOG) * RL_Q + (iidx & (RL_Q - 1))).reshape(B // CH, CH)
    us = jnp.broadcast_to(
        ((uidx >> (RL_LOG - 2)) & 3).astype(jnp.int32).reshape(B, 1), (B, 8))
    isel = jnp.broadcast_to(
        ((iidx >> (RL_LOG - 2)) & 3).astype(jnp.int32).reshape(B, 1), (B, 8))

    utab = _relayout(user_factors.T)
    itab = _relayout(item_factors0.T)
    u2, it2 = _gather(uidx2, iidx2, utab, itab)

    w1u = W1[:, :D].T
    w1i = W1[:, D:].T
    w1u = jnp.pad(w1u, ((0, 0), (0, N_PAD - N_ACT)))
    w1i = jnp.pad(w1i, ((0, 0), (0, N_PAD - N_ACT)))
    b1p = jnp.broadcast_to(jnp.pad(b1, (0, N_PAD - N_ACT)), (8, N_PAD))
    w2p = jnp.pad(W2.T, ((0, N_PAD - N_ACT), (0, O_PAD - 5)))
    b2p = jnp.broadcast_to(jnp.pad(b2, (0, O_PAD - 5)), (8, O_PAD))

    out = _mlp(u2, it2, us, isel, w1u, w1i, b1p, w2p, b2p)
    return out[:, :5]


# speedup vs baseline: 2.5530x; 1.0197x over previous
import functools

import jax
import jax.numpy as jnp
from jax import lax
from jax.experimental import pallas as pl
from jax.experimental.pallas import tpu as pltpu
from jax.experimental.pallas import tpu_sc as plsc

B = 16384
D = 64
DP = 128
NROWS = 1000000
N_ACT = 300
N_PAD = 384
O_PAD = 8
Y_LO, Y_HI = 0.0, 5.5

_info = plsc.get_sparse_core_info()
NC, NS = _info.num_cores, _info.num_subcores
NW = NC * NS
B_PER_W = B // NW
CH = 128
NCHUNK = B_PER_W // CH

RL_BS = 32768
RL_LOG = 15
RL_Q = RL_BS // 4
RL_GRID = (NROWS + RL_BS - 1) // RL_BS
NQ_PAD = RL_GRID * RL_Q


def _pack_pair(a, b):
    ab = lax.bitcast_convert_type(a.astype(jnp.bfloat16), jnp.uint16)
    bb = lax.bitcast_convert_type(b.astype(jnp.bfloat16), jnp.uint16)
    w = (ab.astype(jnp.uint32) << 16) | bb.astype(jnp.uint32)
    return lax.bitcast_convert_type(w, jnp.float32)


def _relayout_body(inT_ref, out_ref):
    x = inT_ref[...]
    p01 = _pack_pair(x[:, 0 * RL_Q:1 * RL_Q], x[:, 1 * RL_Q:2 * RL_Q])
    p23 = _pack_pair(x[:, 2 * RL_Q:3 * RL_Q], x[:, 3 * RL_Q:4 * RL_Q])
    out_ref[:, :D] = p01.T
    out_ref[:, D:] = p23.T


def _relayout(tabT):
    return pl.pallas_call(
        _relayout_body,
        grid=(RL_GRID,),
        in_specs=[pl.BlockSpec((D, RL_BS), lambda i: (0, i))],
        out_specs=pl.BlockSpec((RL_Q, DP), lambda i: (i, 0)),
        out_shape=jax.ShapeDtypeStruct((NQ_PAD, DP), jnp.float32),
    )(tabT)


def _make_gather():
    mesh = plsc.VectorSubcoreMesh(core_axis_name="c", subcore_axis_name="s")

    @functools.partial(
        pl.kernel,
        mesh=mesh,
        compiler_params=pltpu.CompilerParams(use_tc_tiling_on_sc=True),
        out_type=(
            jax.ShapeDtypeStruct((B, DP), jnp.float32),
            jax.ShapeDtypeStruct((B, DP), jnp.float32),
        ),
        scratch_types=[
            pltpu.VMEM((NCHUNK, CH), jnp.int32),
            pltpu.VMEM((NCHUNK, CH), jnp.int32),
            pltpu.VMEM((B_PER_W, DP), jnp.float32),
            pltpu.SemaphoreType.DMA,
        ],
    )
    def gather(uidx_hbm, iidx_hbm, user_hbm, item_hbm, u_out, it_out,
               uidx_v, iidx_v, rows, sem):
        wid = lax.axis_index("s") * NC + lax.axis_index("c")
        base = wid * NCHUNK
        rbase = wid * B_PER_W
        pltpu.sync_copy(uidx_hbm.at[pl.ds(base, NCHUNK)], uidx_v)
        pltpu.sync_copy(iidx_hbm.at[pl.ds(base, NCHUNK)], iidx_v)
        copies = []
        for j in range(NCHUNK):
            copies.append(pltpu.async_copy(
                user_hbm.at[uidx_v.at[j]], rows.at[pl.ds(j * CH, CH)], sem))
        for c in copies:
            c.wait()
        pltpu.sync_copy(rows, u_out.at[pl.ds(rbase, B_PER_W)])
        copies = []
        for j in range(NCHUNK):
            copies.append(pltpu.async_copy(
                item_hbm.at[iidx_v.at[j]], rows.at[pl.ds(j * CH, CH)], sem))
        for c in copies:
            c.wait()
        pltpu.sync_copy(rows, it_out.at[pl.ds(rbase, B_PER_W)])

    return gather


_gather = _make_gather()


def _unpack_select(u2, half, hi):
    w = jnp.where(half > 0, u2[:, D:], u2[:, :D])
    bits = lax.bitcast_convert_type(w, jnp.uint32)
    b16 = jnp.where(hi > 0, bits & jnp.uint32(0xFFFF0000), bits << 16)
    return lax.bitcast_convert_type(b16, jnp.float32)


def _mlp_body(u2_ref, it2_ref, us_ref, is_ref,
              w1u_ref, w1i_ref, b1_ref, w2_ref, b2_ref, out_ref):
    us = us_ref[:, 0:1]
    isel = is_ref[:, 0:1]
    u = _unpack_select(u2_ref[...], us & 2, 1 - (us & 1))
    it = _unpack_select(it2_ref[...], isel & 2, 1 - (isel & 1))
    h = jnp.dot(u, w1u_ref[...], preferred_element_type=jnp.float32)
    h = h + jnp.dot(it, w1i_ref[...], preferred_element_type=jnp.float32)
    h = jnp.maximum(h + b1_ref[0:1, :], 0.0)
    o = jnp.dot(h, w2_ref[...], preferred_element_type=jnp.float32)
    o = o + b2_ref[0:1, :]
    out_ref[...] = jax.nn.sigmoid(o) * (Y_HI - Y_LO) + Y_LO


def _mlp(u2, it2, us, isel, w1u, w1i, b1p, w2p, b2p, bs=2048):
    grid = (B // bs,)
    return pl.pallas_call(
        _mlp_body,
        grid=grid,
        in_specs=[
            pl.BlockSpec((bs, DP), lambda i: (i, 0)),
            pl.BlockSpec((bs, DP), lambda i: (i, 0)),
            pl.BlockSpec((bs, 8), lambda i: (i, 0)),
            pl.BlockSpec((bs, 8), lambda i: (i, 0)),
            pl.BlockSpec((D, N_PAD), lambda i: (0, 0)),
            pl.BlockSpec((D, N_PAD), lambda i: (0, 0)),
            pl.BlockSpec((8, N_PAD), lambda i: (0, 0)),
            pl.BlockSpec((N_PAD, O_PAD), lambda i: (0, 0)),
            pl.BlockSpec((8, O_PAD), lambda i: (0, 0)),
        ],
        out_specs=pl.BlockSpec((bs, O_PAD), lambda i: (i, 0)),
        out_shape=jax.ShapeDtypeStruct((B, O_PAD), jnp.float32),
    )(u2, it2, us, isel, w1u, w1i, b1p, w2p, b2p)


@jax.jit
def kernel(x, user_factors, item_factors0, W1, b1, W2, b2):
    uidx = x[:, 0]
    iidx = x[:, 1]
    uidx2 = ((uidx >> RL_LOG) * RL_Q + (uidx & (RL_Q - 1))).reshape(B // CH, CH)
    iidx2 = ((iidx >> RL_LOG) * RL_Q + (iidx & (RL_Q - 1))).reshape(B // CH, CH)
    us = jnp.broadcast_to(
        ((uidx >> (RL_LOG - 2)) & 3).astype(jnp.int32).reshape(B, 1), (B, 8))
    isel = jnp.broadcast_to(
        ((iidx >> (RL_LOG - 2)) & 3).astype(jnp.int32).reshape(B, 1), (B, 8))

    utab = _relayout(user_factors.T)
    itab = _relayout(item_factors0.T)
    u2, it2 = _gather(uidx2, iidx2, utab, itab)

    w1u = W1[:, :D].T
    w1i = W1[:, D:].T
    w1u = jnp.pad(w1u, ((0, 0), (0, N_PAD - N_ACT)))
    w1i = jnp.pad(w1i, ((0, 0), (0, N_PAD - N_ACT)))
    b1p = jnp.broadcast_to(jnp.pad(b1, (0, N_PAD - N_ACT)), (8, N_PAD))
    w2p = jnp.pad(W2.T, ((0, N_PAD - N_ACT), (0, O_PAD - 5)))
    b2p = jnp.broadcast_to(jnp.pad(b2, (0, O_PAD - 5)), (8, O_PAD))

    out = _mlp(u2, it2, us, isel, w1u, w1i, b1p, w2p, b2p)
    return out[:, :5]
